# baseline (speedup 1.0000x reference)
import jax
import jax.numpy as jnp
import numpy as np
from jax import lax
from jax.experimental import pallas as pl
from jax.experimental.pallas import tpu as pltpu
from jax.experimental.pallas import tpu_sc as plsc

_A_IDX = [0, 0, 0, 0, 0, 0, 1, 1, 1, 1, 1, 2, 2, 2, 2, 1, 1, 1, 0, 0, 0]
_B_IDX = [0, 0, 0, 0, 0, 0, 1, 1, 1, 1, 1, 2, 2, 2, 2, 2, 2, 2, 2, 2, 1]
_C_IDX = [0, 1, 2, 1, 0, 0, 1, 2, 1, 0, 0, 2, 1, 0, 0, 1, 0, 0, 0, 0, 0]
_D_IDX = [0, 1, 2, 2, 2, 1, 1, 2, 2, 2, 1, 2, 2, 2, 1, 2, 2, 1, 2, 1, 1]


def _tables():
    flat = [27 * a + 9 * b + 3 * c + d
            for a, b, c, d in zip(_A_IDX, _B_IDX, _C_IDX, _D_IDX)]
    rows, cols = np.triu_indices(6)
    s2 = np.sqrt(2)
    m = np.array([[1, 1, 1, s2, s2, s2],
                  [1, 1, 1, s2, s2, s2],
                  [1, 1, 1, s2, s2, s2],
                  [s2, s2, s2, 2, 2, 2],
                  [s2, s2, s2, 2, 2, 2],
                  [s2, s2, s2, 2, 2, 2]], dtype=np.float32)
    comp_of = {}
    for k, (r, c) in enumerate(zip(rows, cols)):
        comp_of[(r, c)] = k
        comp_of[(c, r)] = k
    out_comp = [comp_of[(i, j)] for i in range(6) for j in range(6)]
    out_scale = [float(m[i, j]) for i in range(6) for j in range(6)]
    return flat, out_comp, out_scale

_FLAT, _OUT_COMP, _OUT_SCALE = _tables()

_NB = 500000
_S = 160
_NCHUNK = _NB // _S
_NW = 32
_ITERS = -(-_NCHUNK // _NW)


def _body(c_hbm, out_hbm, in0, in1, ou0, ou1, si0, si1, so0, so1):
    wid = lax.axis_index("s") * 2 + lax.axis_index("c")
    lane = lax.iota(jnp.int32, 16)

    ins, outs = (in0, in1), (ou0, ou1)
    isems, osems = (si0, si1), (so0, so1)

    def in_dma(m, slot):
        base = (wid + m * _NW) * _S
        return pltpu.async_copy(c_hbm.at[pl.ds(base, _S)], ins[slot],
                                isems[slot])

    def out_dma(m, slot):
        base = (wid + m * _NW) * _S
        return pltpu.async_copy(outs[slot], out_hbm.at[pl.ds(base, _S)],
                                osems[slot])

    in_dma(0, 0)

    def iter_body(i, _):
        for b in range(2):
            m = 2 * i + b
            chunk = wid + m * _NW
            valid = chunk < _NCHUNK

            @pl.when(valid)
            def _():
                pltpu.make_async_copy(
                    c_hbm.at[pl.ds(chunk * _S, _S)], ins[b], isems[b]).wait()

            @pl.when(wid + (m + 1) * _NW < _NCHUNK)
            def _():
                in_dma(m + 1, 1 - b)

            @pl.when(valid & (m >= 2))
            def _():
                base = (chunk - 2 * _NW) * _S
                pltpu.make_async_copy(
                    outs[b], out_hbm.at[pl.ds(base, _S)], osems[b]).wait()

            @pl.when(valid)
            def _():
                def group_step(g, _):
                    sidx = lane + g * 16
                    vals = [plsc.load_gather(
                                ins[b],
                                [sidx, jnp.full((16,), _FLAT[k], jnp.int32)])
                            for k in range(21)]
                    for j in range(36):
                        plsc.store_scatter(
                            outs[b], [sidx, jnp.full((16,), j, jnp.int32)],
                            vals[_OUT_COMP[j]] * _OUT_SCALE[j])
                    return 0

                lax.fori_loop(0, _S // 16, group_step, 0)
                out_dma(m, b)

        return 0

    lax.fori_loop(0, _ITERS // 2, iter_body, 0)

    for m in (_ITERS - 2, _ITERS - 1):
        chunk = wid + m * _NW

        @pl.when(chunk < _NCHUNK)
        def _():
            pltpu.make_async_copy(
                outs[m % 2], out_hbm.at[pl.ds(chunk * _S, _S)],
                osems[m % 2]).wait()


@jax.jit
def kernel(C):
    c2 = C.reshape(_NB, 81)
    mesh = plsc.VectorSubcoreMesh(core_axis_name="c", subcore_axis_name="s")
    out = pl.kernel(
        _body,
        out_type=jax.ShapeDtypeStruct((_NB, 36), jnp.float32),
        mesh=mesh,
        scratch_types=[
            pltpu.VMEM((_S, 81), jnp.float32),
            pltpu.VMEM((_S, 81), jnp.float32),
            pltpu.VMEM((_S, 36), jnp.float32),
            pltpu.VMEM((_S, 36), jnp.float32),
            pltpu.SemaphoreType.DMA,
            pltpu.SemaphoreType.DMA,
            pltpu.SemaphoreType.DMA,
            pltpu.SemaphoreType.DMA,
        ],
        compiler_params=pltpu.CompilerParams(needs_layout_passes=False,
                                             use_tc_tiling_on_sc=False),
    )(c2)
    return out.reshape(_NB, 6, 6)


# speedup vs baseline: 3.6333x; 3.6333x over previous
import jax
import jax.numpy as jnp
import numpy as np
from jax import lax
from jax.experimental import pallas as pl
from jax.experimental.pallas import tpu as pltpu
from jax.experimental.pallas import tpu_sc as plsc

_A_IDX = [0, 0, 0, 0, 0, 0, 1, 1, 1, 1, 1, 2, 2, 2, 2, 1, 1, 1, 0, 0, 0]
_B_IDX = [0, 0, 0, 0, 0, 0, 1, 1, 1, 1, 1, 2, 2, 2, 2, 2, 2, 2, 2, 2, 1]
_C_IDX = [0, 1, 2, 1, 0, 0, 1, 2, 1, 0, 0, 2, 1, 0, 0, 1, 0, 0, 0, 0, 0]
_D_IDX = [0, 1, 2, 2, 2, 1, 1, 2, 2, 2, 1, 2, 2, 2, 1, 2, 2, 1, 2, 1, 1]


def _tables():
    flat = [27 * a + 9 * b + 3 * c + d
            for a, b, c, d in zip(_A_IDX, _B_IDX, _C_IDX, _D_IDX)]
    rows, cols = np.triu_indices(6)
    s2 = np.sqrt(2)
    m = np.array([[1, 1, 1, s2, s2, s2],
                  [1, 1, 1, s2, s2, s2],
                  [1, 1, 1, s2, s2, s2],
                  [s2, s2, s2, 2, 2, 2],
                  [s2, s2, s2, 2, 2, 2],
                  [s2, s2, s2, 2, 2, 2]], dtype=np.float32)
    comp_of = {}
    for k, (r, c) in enumerate(zip(rows, cols)):
        comp_of[(r, c)] = k
        comp_of[(c, r)] = k
    scale = [float(m[r, c]) for r, c in zip(rows, cols)]
    out_comp = [comp_of[(i, j)] for i in range(6) for j in range(6)]
    return flat, scale, out_comp

_FLAT, _SCALE, _OUT_COMP = _tables()
_NK = 21
_TRS = sorted({r // 8 for r in _FLAT})
_NT = len(_TRS)
_SLAB = [_TRS.index(r // 8) for r in _FLAT]
_ROW = [r % 8 for r in _FLAT]

_NB = 500000
_W = 512
_BP = -(-_NB // _W) * _W
_NCHUNK = _BP // _W
_NW = 32
_ITERS = -(-_NCHUNK // _NW)
_ITERS_2 = -(-_ITERS // 2)


def _body(c_hbm, o_hbm, *rest):
    ins = (rest[:_NT], rest[_NT:2 * _NT])
    base = 2 * _NT
    outs = (rest[base:base + 6], rest[base + 6:base + 12])
    isems = rest[base + 12:base + 14]
    osems = rest[base + 14:base + 16]

    wid = lax.axis_index("s") * 2 + lax.axis_index("c")

    def in_copies(m, slot):
        n0 = (wid + m * _NW) * _W
        return [pltpu.make_async_copy(
                    c_hbm.at[pl.ds(_TRS[t] * 8, 8), pl.ds(n0, _W)],
                    ins[slot][t], isems[slot])
                for t in range(_NT)]

    def out_copies(m, slot):
        n0 = (wid + m * _NW) * _W
        return [pltpu.make_async_copy(
                    outs[slot][i], o_hbm.at[i, :, pl.ds(n0, _W)],
                    osems[slot])
                for i in range(6)]

    for cp in in_copies(0, 0):
        cp.start()

    def iter_body(it, _):
        for b in range(2):
            m = 2 * it + b
            chunk = wid + m * _NW
            valid = chunk < _NCHUNK

            @pl.when(valid)
            def _():
                for cp in in_copies(m, b):
                    cp.wait()

            @pl.when(wid + (m + 1) * _NW < _NCHUNK)
            def _():
                for cp in in_copies(m + 1, 1 - b):
                    cp.start()

            @pl.when(valid & (m >= 2))
            def _():
                for cp in out_copies(m - 2, b):
                    cp.wait()

            @pl.when(valid)
            def _():
                def rep_step(g, _):
                    o = g * 16
                    vals = []
                    for k in range(_NK):
                        v = ins[b][_SLAB[k]][_ROW[k], pl.ds(o, 16)]
                        if _SCALE[k] != 1.0:
                            v = v * _SCALE[k]
                        vals.append(v)
                    for j36 in range(36):
                        i, j = divmod(j36, 6)
                        outs[b][i][j, pl.ds(o, 16)] = vals[_OUT_COMP[j36]]
                    return 0

                lax.fori_loop(0, _W // 16, rep_step, 0)
                for cp in out_copies(m, b):
                    cp.start()

        return 0

    lax.fori_loop(0, _ITERS_2, iter_body, 0)

    for m in (2 * _ITERS_2 - 2, 2 * _ITERS_2 - 1):
        chunk = wid + m * _NW

        @pl.when(chunk < _NCHUNK)
        def _():
            for cp in out_copies(m, m % 2):
                cp.wait()


@jax.jit
def kernel(C):
    c2 = jnp.transpose(C, (1, 2, 3, 4, 0)).reshape(81, _NB)
    c_t = jnp.pad(c2, ((0, 7), (0, _BP - _NB)))
    mesh = plsc.VectorSubcoreMesh(core_axis_name="c", subcore_axis_name="s")
    scratch = [pltpu.VMEM((8, _W), jnp.float32) for _ in range(2 * _NT)]
    scratch += [pltpu.VMEM((6, _W), jnp.float32) for _ in range(12)]
    scratch += [pltpu.SemaphoreType.DMA] * 4
    o_t = pl.kernel(
        _body,
        out_type=jax.ShapeDtypeStruct((6, 6, _BP), jnp.float32),
        mesh=mesh,
        scratch_types=scratch,
        compiler_params=pltpu.CompilerParams(needs_layout_passes=False),
    )(c_t)
    return jnp.transpose(o_t[:, :, :_NB], (2, 0, 1))


# speedup vs baseline: 12.1706x; 3.3497x over previous
import jax
import jax.numpy as jnp
import numpy as np
from jax import lax
from jax.experimental import pallas as pl
from jax.experimental.pallas import tpu as pltpu
from jax.experimental.pallas import tpu_sc as plsc

_A_IDX = [0, 0, 0, 0, 0, 0, 1, 1, 1, 1, 1, 2, 2, 2, 2, 1, 1, 1, 0, 0, 0]
_B_IDX = [0, 0, 0, 0, 0, 0, 1, 1, 1, 1, 1, 2, 2, 2, 2, 2, 2, 2, 2, 2, 1]
_C_IDX = [0, 1, 2, 1, 0, 0, 1, 2, 1, 0, 0, 2, 1, 0, 0, 1, 0, 0, 0, 0, 0]
_D_IDX = [0, 1, 2, 2, 2, 1, 1, 2, 2, 2, 1, 2, 2, 2, 1, 2, 2, 1, 2, 1, 1]


def _tables():
    flat = [27 * a + 9 * b + 3 * c + d
            for a, b, c, d in zip(_A_IDX, _B_IDX, _C_IDX, _D_IDX)]
    rows, cols = np.triu_indices(6)
    s2 = np.sqrt(2)
    m = np.array([[1, 1, 1, s2, s2, s2],
                  [1, 1, 1, s2, s2, s2],
                  [1, 1, 1, s2, s2, s2],
                  [s2, s2, s2, 2, 2, 2],
                  [s2, s2, s2, 2, 2, 2],
                  [s2, s2, s2, 2, 2, 2]], dtype=np.float32)
    comp_of = {}
    for k, (r, c) in enumerate(zip(rows, cols)):
        comp_of[(r, c)] = k
        comp_of[(c, r)] = k
    scale = [float(m[r, c]) for r, c in zip(rows, cols)]
    out_comp = [comp_of[(i, j)] for i in range(6) for j in range(6)]
    return flat, scale, out_comp, m

_FLAT, _SCALE, _OUT_COMP, _MASK = _tables()
_NK = 21
_ABCS = sorted({f // 3 for f in _FLAT})
_NG = len(_ABCS)
_GRP = [_ABCS.index(f // 3) for f in _FLAT]
_ROW = [f % 3 for f in _FLAT]

_NB = 500000
_W = 384
_NCHUNK = _NB // _W
_TAIL = _NB - _NCHUNK * _W
_BP = _NCHUNK * _W + 128
_NW = 32
_ITERS = -(-_NCHUNK // _NW)
_ITERS_2 = -(-_ITERS // 2)


def _body(c_hbm, o_hbm, *rest):
    ins = (rest[:_NG], rest[_NG:2 * _NG])
    base = 2 * _NG
    outs = (rest[base:base + 6], rest[base + 6:base + 12])
    isems = rest[base + 12:base + 14]
    osems = rest[base + 14:base + 16]

    wid = lax.axis_index("s") * 2 + lax.axis_index("c")

    def in_copies(m, slot):
        n0 = (wid + m * _NW) * _W
        cps = []
        for g in range(_NG):
            a, b, c = np.unravel_index(_ABCS[g], (3, 3, 3))
            cps.append(pltpu.make_async_copy(
                c_hbm.at[int(a), int(b), int(c), :, pl.ds(n0, _W)],
                ins[slot][g], isems[slot]))
        return cps

    def out_copies(m, slot):
        n0 = (wid + m * _NW) * _W
        return [pltpu.make_async_copy(
                    outs[slot][i], o_hbm.at[i, :, pl.ds(n0, _W)],
                    osems[slot])
                for i in range(6)]

    for cp in in_copies(0, 0):
        cp.start()

    def iter_body(it, _):
        for b in range(2):
            m = 2 * it + b
            chunk = wid + m * _NW
            valid = chunk < _NCHUNK

            @pl.when(valid)
            def _():
                for cp in in_copies(m, b):
                    cp.wait()

            @pl.when(wid + (m + 1) * _NW < _NCHUNK)
            def _():
                for cp in in_copies(m + 1, 1 - b):
                    cp.start()

            @pl.when(valid & (m >= 2))
            def _():
                for cp in out_copies(m - 2, b):
                    cp.wait()

            @pl.when(valid)
            def _():
                def rep_step(g, _):
                    o = g * 16
                    vals = []
                    for k in range(_NK):
                        v = ins[b][_GRP[k]][_ROW[k], pl.ds(o, 16)]
                        if _SCALE[k] != 1.0:
                            v = v * _SCALE[k]
                        vals.append(v)
                    for j36 in range(36):
                        i, j = divmod(j36, 6)
                        outs[b][i][j, pl.ds(o, 16)] = vals[_OUT_COMP[j36]]
                    return 0

                lax.fori_loop(0, _W // 16, rep_step, 0)
                for cp in out_copies(m, b):
                    cp.start()

        return 0

    lax.fori_loop(0, _ITERS_2, iter_body, 0)

    for m in (2 * _ITERS_2 - 2, 2 * _ITERS_2 - 1):
        chunk = wid + m * _NW

        @pl.when(chunk < _NCHUNK)
        def _():
            for cp in out_copies(m, m % 2):
                cp.wait()


@jax.jit
def kernel(C):
    c5 = jnp.transpose(C, (1, 2, 3, 4, 0))
    mesh = plsc.VectorSubcoreMesh(core_axis_name="c", subcore_axis_name="s")
    scratch = [pltpu.VMEM((3, _W), jnp.float32) for _ in range(2 * _NG)]
    scratch += [pltpu.VMEM((6, _W), jnp.float32) for _ in range(12)]
    scratch += [pltpu.SemaphoreType.DMA] * 4
    o_t = pl.kernel(
        _body,
        out_type=jax.ShapeDtypeStruct((6, 6, _BP), jnp.float32),
        mesh=mesh,
        scratch_types=scratch,
        compiler_params=pltpu.CompilerParams(needs_layout_passes=False),
    )(c5)
    ta = jnp.asarray([_A_IDX[k] for k in _OUT_COMP])
    tb = jnp.asarray([_B_IDX[k] for k in _OUT_COMP])
    tc = jnp.asarray([_C_IDX[k] for k in _OUT_COMP])
    td = jnp.asarray([_D_IDX[k] for k in _OUT_COMP])
    tail = C[_NCHUNK * _W:]
    tv = tail[:, ta, tb, tc, td] * jnp.asarray(_MASK.reshape(36))
    o_t = lax.dynamic_update_slice(
        o_t, jnp.transpose(tv, (1, 0)).reshape(6, 6, _TAIL),
        (0, 0, _NCHUNK * _W))
    return jnp.transpose(o_t[:, :, :_NB], (2, 0, 1))


# speedup vs baseline: 19.6246x; 1.6125x over previous
import jax
import jax.numpy as jnp
import numpy as np
from jax import lax
from jax.experimental import pallas as pl
from jax.experimental.pallas import tpu as pltpu
from jax.experimental.pallas import tpu_sc as plsc

_A_IDX = [0, 0, 0, 0, 0, 0, 1, 1, 1, 1, 1, 2, 2, 2, 2, 1, 1, 1, 0, 0, 0]
_B_IDX = [0, 0, 0, 0, 0, 0, 1, 1, 1, 1, 1, 2, 2, 2, 2, 2, 2, 2, 2, 2, 1]
_C_IDX = [0, 1, 2, 1, 0, 0, 1, 2, 1, 0, 0, 2, 1, 0, 0, 1, 0, 0, 0, 0, 0]
_D_IDX = [0, 1, 2, 2, 2, 1, 1, 2, 2, 2, 1, 2, 2, 2, 1, 2, 2, 1, 2, 1, 1]


def _tables():
    flat = [27 * a + 9 * b + 3 * c + d
            for a, b, c, d in zip(_A_IDX, _B_IDX, _C_IDX, _D_IDX)]
    rows, cols = np.triu_indices(6)
    s2 = np.sqrt(2)
    m = np.array([[1, 1, 1, s2, s2, s2],
                  [1, 1, 1, s2, s2, s2],
                  [1, 1, 1, s2, s2, s2],
                  [s2, s2, s2, 2, 2, 2],
                  [s2, s2, s2, 2, 2, 2],
                  [s2, s2, s2, 2, 2, 2]], dtype=np.float32)
    comp_of = {}
    for k, (r, c) in enumerate(zip(rows, cols)):
        comp_of[(r, c)] = k
        comp_of[(c, r)] = k
    scale = [float(m[r, c]) for r, c in zip(rows, cols)]
    out_comp = [comp_of[(i, j)] for i in range(6) for j in range(6)]
    return flat, scale, out_comp, m

_FLAT, _SCALE, _OUT_COMP, _MASK = _tables()
_NK = 21
_ABCS = sorted({f // 3 for f in _FLAT})
_NG = len(_ABCS)
_GRP = [_ABCS.index(f // 3) for f in _FLAT]
_ROW = [f % 3 for f in _FLAT]

_NB = 500000
_W = 384
_NCHUNK = _NB // _W
_TAIL = _NB - _NCHUNK * _W
_NW = 32
_ITERS = -(-_NCHUNK // _NW)
_ITERS_2 = -(-_ITERS // 2)


def _body(c_hbm, o_hbm, *rest):
    ins = (rest[:_NG], rest[_NG:2 * _NG])
    base = 2 * _NG
    outs = (rest[base:base + 6], rest[base + 6:base + 12])
    isems = rest[base + 12:base + 14]
    osems = rest[base + 14:base + 16]

    wid = lax.axis_index("s") * 2 + lax.axis_index("c")

    def in_copies(m, slot):
        n0 = (wid + m * _NW) * _W
        cps = []
        for g in range(_NG):
            a, b, c = np.unravel_index(_ABCS[g], (3, 3, 3))
            cps.append(pltpu.make_async_copy(
                c_hbm.at[int(a), int(b), int(c), :, pl.ds(n0, _W)],
                ins[slot][g], isems[slot]))
        return cps

    def out_copies(m, slot):
        n0 = (wid + m * _NW) * _W
        return [pltpu.make_async_copy(
                    outs[slot][i], o_hbm.at[i, :, pl.ds(n0, _W)],
                    osems[slot])
                for i in range(6)]

    for cp in in_copies(0, 0):
        cp.start()

    def iter_body(it, _):
        for b in range(2):
            m = 2 * it + b
            chunk = wid + m * _NW
            valid = chunk < _NCHUNK

            @pl.when(valid)
            def _():
                for cp in in_copies(m, b):
                    cp.wait()

            @pl.when(wid + (m + 1) * _NW < _NCHUNK)
            def _():
                for cp in in_copies(m + 1, 1 - b):
                    cp.start()

            @pl.when(valid & (m >= 2))
            def _():
                for cp in out_copies(m - 2, b):
                    cp.wait()

            @pl.when(valid)
            def _():
                def rep_step(g, _):
                    o = g * 16
                    vals = []
                    for k in range(_NK):
                        v = ins[b][_GRP[k]][_ROW[k], pl.ds(o, 16)]
                        if _SCALE[k] != 1.0:
                            v = v * _SCALE[k]
                        vals.append(v)
                    for j36 in range(36):
                        i, j = divmod(j36, 6)
                        outs[b][i][j, pl.ds(o, 16)] = vals[_OUT_COMP[j36]]
                    return 0

                lax.fori_loop(0, _W // 16, rep_step, 0)
                for cp in out_copies(m, b):
                    cp.start()

        return 0

    lax.fori_loop(0, _ITERS_2, iter_body, 0)

    for m in (2 * _ITERS_2 - 2, 2 * _ITERS_2 - 1):
        chunk = wid + m * _NW

        @pl.when(chunk < _NCHUNK)
        def _():
            for cp in out_copies(m, m % 2):
                cp.wait()


@jax.jit
def kernel(C):
    c5 = jnp.transpose(C, (1, 2, 3, 4, 0))
    mesh = plsc.VectorSubcoreMesh(core_axis_name="c", subcore_axis_name="s")
    scratch = [pltpu.VMEM((3, _W), jnp.float32) for _ in range(2 * _NG)]
    scratch += [pltpu.VMEM((6, _W), jnp.float32) for _ in range(12)]
    scratch += [pltpu.SemaphoreType.DMA] * 4
    o_t = pl.kernel(
        _body,
        out_type=jax.ShapeDtypeStruct((6, 6, _NB), jnp.float32),
        mesh=mesh,
        scratch_types=scratch,
        compiler_params=pltpu.CompilerParams(needs_layout_passes=False),
    )(c5)
    ta = jnp.asarray([_A_IDX[k] for k in _OUT_COMP])
    tb = jnp.asarray([_B_IDX[k] for k in _OUT_COMP])
    tc = jnp.asarray([_C_IDX[k] for k in _OUT_COMP])
    td = jnp.asarray([_D_IDX[k] for k in _OUT_COMP])
    tail = C[_NCHUNK * _W:]
    tv = tail[:, ta, tb, tc, td] * jnp.asarray(_MASK.reshape(36))
    o_t = lax.dynamic_update_slice(
        o_t, jnp.transpose(tv, (1, 0)).reshape(6, 6, _TAIL),
        (0, 0, _NCHUNK * _W))
    return jnp.transpose(o_t, (2, 0, 1))


# speedup vs baseline: 19.7190x; 1.0048x over previous
import jax
import jax.numpy as jnp
import numpy as np
from jax import lax
from jax.experimental import pallas as pl
from jax.experimental.pallas import tpu as pltpu
from jax.experimental.pallas import tpu_sc as plsc

_A_IDX = [0, 0, 0, 0, 0, 0, 1, 1, 1, 1, 1, 2, 2, 2, 2, 1, 1, 1, 0, 0, 0]
_B_IDX = [0, 0, 0, 0, 0, 0, 1, 1, 1, 1, 1, 2, 2, 2, 2, 2, 2, 2, 2, 2, 1]
_C_IDX = [0, 1, 2, 1, 0, 0, 1, 2, 1, 0, 0, 2, 1, 0, 0, 1, 0, 0, 0, 0, 0]
_D_IDX = [0, 1, 2, 2, 2, 1, 1, 2, 2, 2, 1, 2, 2, 2, 1, 2, 2, 1, 2, 1, 1]


def _tables():
    flat = [27 * a + 9 * b + 3 * c + d
            for a, b, c, d in zip(_A_IDX, _B_IDX, _C_IDX, _D_IDX)]
    rows, cols = np.triu_indices(6)
    s2 = np.sqrt(2)
    m = np.array([[1, 1, 1, s2, s2, s2],
                  [1, 1, 1, s2, s2, s2],
                  [1, 1, 1, s2, s2, s2],
                  [s2, s2, s2, 2, 2, 2],
                  [s2, s2, s2, 2, 2, 2],
                  [s2, s2, s2, 2, 2, 2]], dtype=np.float32)
    comp_of = {}
    for k, (r, c) in enumerate(zip(rows, cols)):
        comp_of[(r, c)] = k
        comp_of[(c, r)] = k
    scale = [float(m[r, c]) for r, c in zip(rows, cols)]
    out_comp = [comp_of[(i, j)] for i in range(6) for j in range(6)]
    return flat, scale, out_comp, m

_FLAT, _SCALE, _OUT_COMP, _MASK = _tables()
_NK = 21
_SLABS = [(0, 0, 0, 3), (0, 1, 0, 1), (0, 2, 0, 1),
          (1, 1, 0, 3), (1, 2, 0, 2), (2, 2, 0, 3)]
_NG = len(_SLABS)


def _slab_of(f):
    ab, c, d = f // 9, (f // 3) % 3, f % 3
    a, b = ab // 3, ab % 3
    for s, (sa, sb, c0, nc) in enumerate(_SLABS):
        if sa == a and sb == b and c0 <= c < c0 + nc:
            return s, c - c0, d
    raise AssertionError(f)

_GRP = [_slab_of(f) for f in _FLAT]

_NB = 500000
_W = 384
_NCHUNK = _NB // _W
_TAIL = _NB - _NCHUNK * _W
_NW = 32
_ITERS = -(-_NCHUNK // _NW)
_ITERS_2 = -(-_ITERS // 2)


def _body(c_hbm, o_hbm, *rest):
    ins = (rest[:_NG], rest[_NG:2 * _NG])
    base = 2 * _NG
    outs = (rest[base:base + 6], rest[base + 6:base + 12])
    isems = rest[base + 12:base + 14]
    osems = rest[base + 14:base + 16]

    wid = lax.axis_index("s") * 2 + lax.axis_index("c")

    def in_copies(m, slot):
        n0 = (wid + m * _NW) * _W
        cps = []
        for g, (a, b, c0, nc) in enumerate(_SLABS):
            cps.append(pltpu.make_async_copy(
                c_hbm.at[a, b, pl.ds(c0, nc), :, pl.ds(n0, _W)],
                ins[slot][g], isems[slot]))
        return cps

    def out_copies(m, slot):
        n0 = (wid + m * _NW) * _W
        return [pltpu.make_async_copy(
                    outs[slot][i], o_hbm.at[i, :, pl.ds(n0, _W)],
                    osems[slot])
                for i in range(6)]

    for cp in in_copies(0, 0):
        cp.start()

    def iter_body(it, _):
        for b in range(2):
            m = 2 * it + b
            chunk = wid + m * _NW
            valid = chunk < _NCHUNK

            @pl.when(valid)
            def _():
                for cp in in_copies(m, b):
                    cp.wait()

            @pl.when(wid + (m + 1) * _NW < _NCHUNK)
            def _():
                for cp in in_copies(m + 1, 1 - b):
                    cp.start()

            @pl.when(valid & (m >= 2))
            def _():
                for cp in out_copies(m - 2, b):
                    cp.wait()

            @pl.when(valid)
            def _():
                @plsc.parallel_loop(0, _W // 16, 1, unroll=2)
                def rep_step(g):
                    o = g * 16
                    vals = []
                    for k in range(_NK):
                        s, c, d = _GRP[k]
                        v = ins[b][s][c, d, pl.ds(o, 16)]
                        if _SCALE[k] != 1.0:
                            v = v * _SCALE[k]
                        vals.append(v)
                    for j36 in range(36):
                        i, j = divmod(j36, 6)
                        outs[b][i][j, pl.ds(o, 16)] = vals[_OUT_COMP[j36]]

                for cp in out_copies(m, b):
                    cp.start()

        return 0

    lax.fori_loop(0, _ITERS_2, iter_body, 0)

    for m in (2 * _ITERS_2 - 2, 2 * _ITERS_2 - 1):
        chunk = wid + m * _NW

        @pl.when(chunk < _NCHUNK)
        def _():
            for cp in out_copies(m, m % 2):
                cp.wait()


@jax.jit
def kernel(C):
    c5 = jnp.transpose(C, (1, 2, 3, 4, 0))
    mesh = plsc.VectorSubcoreMesh(core_axis_name="c", subcore_axis_name="s")
    scratch = [pltpu.VMEM((nc, 3, _W), jnp.float32)
               for _ in range(2) for (_, _, _, nc) in _SLABS]
    scratch += [pltpu.VMEM((6, _W), jnp.float32) for _ in range(12)]
    scratch += [pltpu.SemaphoreType.DMA] * 4
    o_t = pl.kernel(
        _body,
        out_type=jax.ShapeDtypeStruct((6, 6, _NB), jnp.float32),
        mesh=mesh,
        scratch_types=scratch,
        compiler_params=pltpu.CompilerParams(needs_layout_passes=False),
    )(c5)
    ta = jnp.asarray([_A_IDX[k] for k in _OUT_COMP])
    tb = jnp.asarray([_B_IDX[k] for k in _OUT_COMP])
    tc = jnp.asarray([_C_IDX[k] for k in _OUT_COMP])
    td = jnp.asarray([_D_IDX[k] for k in _OUT_COMP])
    tail = C[_NCHUNK * _W:]
    tv = tail[:, ta, tb, tc, td] * jnp.asarray(_MASK.reshape(36))
    o_t = lax.dynamic_update_slice(
        o_t, jnp.transpose(tv, (1, 0)).reshape(6, 6, _TAIL),
        (0, 0, _NCHUNK * _W))
    return jnp.transpose(o_t, (2, 0, 1))
